# baseline (speedup 1.0000x reference)
import functools

import jax
import jax.numpy as jnp
from jax import lax
from jax.experimental import pallas as pl
from jax.experimental.pallas import tpu as pltpu
from jax.experimental.pallas import tpu_sc as plsc

DIM = 64
BATCH = 16384 * 50
NROWS = 1000000

_info = plsc.get_sparse_core_info()
NUM_CORES = _info.num_cores
NUM_SUBCORES = _info.num_subcores
NUM_WORKERS = NUM_CORES * NUM_SUBCORES

ROWS_PER_WORKER = BATCH // NUM_WORKERS
CHUNK = 400
NUM_CHUNKS = ROWS_PER_WORKER // CHUNK
NBUF = 4
SHIFT = 2
NUM_ROUNDS = NUM_CHUNKS // NBUF

TBLK = 512
OBLK = TBLK // 2
TGRID = (NROWS + TBLK - 1) // TBLK


def _repack_body(in_ref, out_ref):
    out_ref[...] = pltpu.einshape("d(ph)->p(hd)", in_ref[...], h=2)


_repack = pl.pallas_call(
    _repack_body,
    grid=(TGRID,),
    in_specs=[pl.BlockSpec((DIM, TBLK), lambda i: (0, i))],
    out_specs=pl.BlockSpec((OBLK, 128), lambda i: (i, 0)),
    out_shape=jax.ShapeDtypeStruct((NROWS // 2, 128), jnp.float32),
)


@functools.partial(
    pl.kernel,
    out_type=jax.ShapeDtypeStruct((BATCH, DIM), jnp.float32),
    mesh=plsc.VectorSubcoreMesh(core_axis_name="c", subcore_axis_name="s"),
    compiler_params=pltpu.CompilerParams(use_tc_tiling_on_sc=False),
    scratch_types=(
        [pltpu.VMEM((ROWS_PER_WORKER,), jnp.int32)]
        + [pltpu.VMEM((CHUNK, DIM), jnp.float32) for _ in range(NBUF)]
        + [pltpu.SemaphoreType.DMA for _ in range(2 * NBUF)]
    ),
)
def _gather_kernel(table_hbm, idx_hbm, out_hbm, idx_all, *bufs_and_sems):
    rows = bufs_and_sems[:NBUF]
    gsem = bufs_and_sems[NBUF:2 * NBUF]
    wsem = bufs_and_sems[2 * NBUF:]

    wid = lax.axis_index("s") * NUM_CORES + lax.axis_index("c")
    base = wid * ROWS_PER_WORKER
    pltpu.sync_copy(idx_hbm.at[pl.ds(base, ROWS_PER_WORKER)], idx_all)

    def start_gather(g, b):
        pltpu.async_copy(
            table_hbm.at[idx_all.at[pl.ds(g * CHUNK, CHUNK)]], rows[b], gsem[b])

    def wait_gather(g, b):
        pltpu.make_async_copy(
            table_hbm.at[idx_all.at[pl.ds(g * CHUNK, CHUNK)]], rows[b], gsem[b]).wait()

    def start_writeback(g, b):
        pltpu.async_copy(rows[b], out_hbm.at[pl.ds(base + g * CHUNK, CHUNK)], wsem[b])

    def wait_writeback(g, b):
        pltpu.make_async_copy(
            rows[b], out_hbm.at[pl.ds(base + g * CHUNK, CHUNK)], wsem[b]).wait()

    for b in range(NBUF):
        start_gather(b, b)
        if b >= SHIFT:
            wait_gather(b - SHIFT, b - SHIFT)
            start_writeback(b - SHIFT, b - SHIFT)

    def round_body(i, carry):
        g0 = i * NBUF
        for b in range(NBUF):
            g = g0 + b
            wait_writeback(g - NBUF, b)
            start_gather(g, b)
            bp = (b - SHIFT) % NBUF
            wait_gather(g - SHIFT, bp)
            start_writeback(g - SHIFT, bp)
        return carry

    lax.fori_loop(1, NUM_ROUNDS, round_body, 0)

    for j in range(SHIFT):
        g = NUM_CHUNKS - SHIFT + j
        b = g % NBUF
        wait_gather(g, b)
        start_writeback(g, b)
    for b in range(NBUF):
        g = NUM_CHUNKS - NBUF + b
        wait_writeback(g, b)


def kernel(token_ids, weights):
    idx = token_ids.reshape(-1).astype(jnp.int32)
    packed = _repack(weights.T)
    table_lin = packed.reshape(NROWS, DIM)
    out = _gather_kernel(table_lin, idx)
    return out.reshape(token_ids.shape + (DIM,))


# speedup vs baseline: 6.9013x; 6.9013x over previous
import functools

import jax
import jax.numpy as jnp
from jax import lax
from jax.experimental import pallas as pl
from jax.experimental.pallas import tpu as pltpu
from jax.experimental.pallas import tpu_sc as plsc

DIM = 64
BATCH = 16384 * 50
NROWS = 1000000

_info = plsc.get_sparse_core_info()
NUM_CORES = _info.num_cores
NUM_SUBCORES = _info.num_subcores
NUM_WORKERS = NUM_CORES * NUM_SUBCORES

ROWS_PER_WORKER = BATCH // NUM_WORKERS
CHUNK = 400
NUM_CHUNKS = ROWS_PER_WORKER // CHUNK
NBUF = 4
SHIFT = 2
NUM_ROUNDS = NUM_CHUNKS // NBUF

TBLK = 512
TGRID = 977
HALF = TBLK * TGRID


def _repack_body(in1_ref, in2_ref, out_ref):
    out_ref[:, 0:DIM] = in1_ref[...].T
    out_ref[:, DIM:128] = in2_ref[...].T


_repack = pl.pallas_call(
    _repack_body,
    grid=(TGRID,),
    in_specs=[
        pl.BlockSpec((DIM, TBLK), lambda i: (0, i)),
        pl.BlockSpec((DIM, TBLK), lambda i: (0, i + TGRID)),
    ],
    out_specs=pl.BlockSpec((TBLK, 128), lambda i: (i, 0)),
    out_shape=jax.ShapeDtypeStruct((HALF, 128), jnp.float32),
)


@functools.partial(
    pl.kernel,
    out_type=jax.ShapeDtypeStruct((BATCH, DIM), jnp.float32),
    mesh=plsc.VectorSubcoreMesh(core_axis_name="c", subcore_axis_name="s"),
    compiler_params=pltpu.CompilerParams(use_tc_tiling_on_sc=False),
    scratch_types=(
        [pltpu.VMEM((ROWS_PER_WORKER,), jnp.int32)]
        + [pltpu.VMEM((CHUNK, DIM), jnp.float32) for _ in range(NBUF)]
        + [pltpu.SemaphoreType.DMA for _ in range(2 * NBUF)]
    ),
)
def _gather_kernel(table_hbm, idx_hbm, out_hbm, idx_all, *bufs_and_sems):
    rows = bufs_and_sems[:NBUF]
    gsem = bufs_and_sems[NBUF:2 * NBUF]
    wsem = bufs_and_sems[2 * NBUF:]

    wid = lax.axis_index("s") * NUM_CORES + lax.axis_index("c")
    base = wid * ROWS_PER_WORKER
    pltpu.sync_copy(idx_hbm.at[pl.ds(base, ROWS_PER_WORKER)], idx_all)

    def start_gather(g, b):
        pltpu.async_copy(
            table_hbm.at[idx_all.at[pl.ds(g * CHUNK, CHUNK)]], rows[b], gsem[b])

    def wait_gather(g, b):
        pltpu.make_async_copy(
            table_hbm.at[idx_all.at[pl.ds(g * CHUNK, CHUNK)]], rows[b], gsem[b]).wait()

    def start_writeback(g, b):
        pltpu.async_copy(rows[b], out_hbm.at[pl.ds(base + g * CHUNK, CHUNK)], wsem[b])

    def wait_writeback(g, b):
        pltpu.make_async_copy(
            rows[b], out_hbm.at[pl.ds(base + g * CHUNK, CHUNK)], wsem[b]).wait()

    for b in range(NBUF):
        start_gather(b, b)
        if b >= SHIFT:
            wait_gather(b - SHIFT, b - SHIFT)
            start_writeback(b - SHIFT, b - SHIFT)

    def round_body(i, carry):
        g0 = i * NBUF
        for b in range(NBUF):
            g = g0 + b
            wait_writeback(g - NBUF, b)
            start_gather(g, b)
            bp = (b - SHIFT) % NBUF
            wait_gather(g - SHIFT, bp)
            start_writeback(g - SHIFT, bp)
        return carry

    lax.fori_loop(1, NUM_ROUNDS, round_body, 0)

    for j in range(SHIFT):
        g = NUM_CHUNKS - SHIFT + j
        b = g % NBUF
        wait_gather(g, b)
        start_writeback(g, b)
    for b in range(NBUF):
        g = NUM_CHUNKS - NBUF + b
        wait_writeback(g, b)


def kernel(token_ids, weights):
    r = token_ids.reshape(-1).astype(jnp.int32)
    idx = 2 * (r % HALF) + r // HALF
    packed = _repack(weights.T, weights.T)
    table_lin = packed.reshape(2 * HALF, DIM)
    out = _gather_kernel(table_lin, idx)
    return out.reshape(token_ids.shape + (DIM,))


# speedup vs baseline: 9.3189x; 1.3503x over previous
import functools

import jax
import jax.numpy as jnp
from jax import lax
from jax.experimental import pallas as pl
from jax.experimental.pallas import tpu as pltpu
from jax.experimental.pallas import tpu_sc as plsc

DIM = 64
BATCH = 16384 * 50
NROWS = 1000000

_info = plsc.get_sparse_core_info()
NUM_CORES = _info.num_cores
NUM_SUBCORES = _info.num_subcores
NUM_WORKERS = NUM_CORES * NUM_SUBCORES

ROWS_PER_WORKER = BATCH // NUM_WORKERS
CHUNK = 400
NUM_CHUNKS = ROWS_PER_WORKER // CHUNK
NBUF = 4
SHIFT = 2
NUM_ROUNDS = NUM_CHUNKS // NBUF

TBLK = 2048
TGRID = 245
HALF = TBLK * TGRID


def _repack_body(in1_ref, in2_ref, out_ref):
    out_ref[:, 0:DIM] = in1_ref[...].T
    out_ref[:, DIM:128] = in2_ref[...].T


_repack = pl.pallas_call(
    _repack_body,
    grid=(TGRID,),
    in_specs=[
        pl.BlockSpec((DIM, TBLK), lambda i: (0, i)),
        pl.BlockSpec(
            (DIM, TBLK),
            lambda i: (0, jnp.minimum(i + TGRID, (NROWS - 1) // TBLK))),
    ],
    out_specs=pl.BlockSpec((TBLK, 128), lambda i: (i, 0)),
    out_shape=jax.ShapeDtypeStruct((HALF, 128), jnp.float32),
)


@functools.partial(
    pl.kernel,
    out_type=jax.ShapeDtypeStruct((BATCH, DIM), jnp.float32),
    mesh=plsc.VectorSubcoreMesh(core_axis_name="c", subcore_axis_name="s"),
    compiler_params=pltpu.CompilerParams(use_tc_tiling_on_sc=False),
    scratch_types=(
        [pltpu.VMEM((ROWS_PER_WORKER,), jnp.int32)]
        + [pltpu.VMEM((CHUNK, DIM), jnp.float32) for _ in range(NBUF)]
        + [pltpu.SemaphoreType.DMA for _ in range(2 * NBUF)]
    ),
)
def _gather_kernel(table_hbm, idx_hbm, out_hbm, idx_all, *bufs_and_sems):
    rows = bufs_and_sems[:NBUF]
    gsem = bufs_and_sems[NBUF:2 * NBUF]
    wsem = bufs_and_sems[2 * NBUF:]

    wid = lax.axis_index("s") * NUM_CORES + lax.axis_index("c")
    base = wid * ROWS_PER_WORKER
    pltpu.sync_copy(idx_hbm.at[pl.ds(base, ROWS_PER_WORKER)], idx_all)

    def start_gather(g, b):
        pltpu.async_copy(
            table_hbm.at[idx_all.at[pl.ds(g * CHUNK, CHUNK)]], rows[b], gsem[b])

    def wait_gather(g, b):
        pltpu.make_async_copy(
            table_hbm.at[idx_all.at[pl.ds(g * CHUNK, CHUNK)]], rows[b], gsem[b]).wait()

    def start_writeback(g, b):
        pltpu.async_copy(rows[b], out_hbm.at[pl.ds(base + g * CHUNK, CHUNK)], wsem[b])

    def wait_writeback(g, b):
        pltpu.make_async_copy(
            rows[b], out_hbm.at[pl.ds(base + g * CHUNK, CHUNK)], wsem[b]).wait()

    for b in range(NBUF):
        start_gather(b, b)
        if b >= SHIFT:
            wait_gather(b - SHIFT, b - SHIFT)
            start_writeback(b - SHIFT, b - SHIFT)

    def round_body(i, carry):
        g0 = i * NBUF
        for b in range(NBUF):
            g = g0 + b
            wait_writeback(g - NBUF, b)
            start_gather(g, b)
            bp = (b - SHIFT) % NBUF
            wait_gather(g - SHIFT, bp)
            start_writeback(g - SHIFT, bp)
        return carry

    lax.fori_loop(1, NUM_ROUNDS, round_body, 0)

    for j in range(SHIFT):
        g = NUM_CHUNKS - SHIFT + j
        b = g % NBUF
        wait_gather(g, b)
        start_writeback(g, b)
    for b in range(NBUF):
        g = NUM_CHUNKS - NBUF + b
        wait_writeback(g, b)


def kernel(token_ids, weights):
    r = token_ids.reshape(-1).astype(jnp.int32)
    idx = 2 * (r % HALF) + r // HALF
    packed = _repack(weights.T, weights.T)
    table_lin = packed.reshape(2 * HALF, DIM)
    out = _gather_kernel(table_lin, idx)
    return out.reshape(token_ids.shape + (DIM,))


# speedup vs baseline: 9.9770x; 1.0706x over previous
import functools

import jax
import jax.numpy as jnp
from jax import lax
from jax.experimental import pallas as pl
from jax.experimental.pallas import tpu as pltpu
from jax.experimental.pallas import tpu_sc as plsc

DIM = 64
BATCH = 16384 * 50
NROWS = 1000000

_info = plsc.get_sparse_core_info()
NUM_CORES = _info.num_cores
NUM_SUBCORES = _info.num_subcores
NUM_WORKERS = NUM_CORES * NUM_SUBCORES

ROWS_PER_WORKER = BATCH // NUM_WORKERS
CHUNK = 400
NUM_CHUNKS = ROWS_PER_WORKER // CHUNK
NBUF = 4
SHIFT = 2
NUM_ROUNDS = NUM_CHUNKS // NBUF

TBLK = 4096
TGRID = 123
HALF = TBLK * TGRID


def _repack_body(in1_ref, in2_ref, out_ref):
    out_ref[:, 0:DIM] = in1_ref[...].T
    out_ref[:, DIM:128] = in2_ref[...].T


_repack = pl.pallas_call(
    _repack_body,
    grid=(TGRID,),
    in_specs=[
        pl.BlockSpec((DIM, TBLK), lambda i: (0, i)),
        pl.BlockSpec(
            (DIM, TBLK),
            lambda i: (0, jnp.minimum(i + TGRID, (NROWS - 1) // TBLK))),
    ],
    out_specs=pl.BlockSpec((TBLK, 128), lambda i: (i, 0)),
    out_shape=jax.ShapeDtypeStruct((HALF, 128), jnp.float32),
)


@functools.partial(
    pl.kernel,
    out_type=jax.ShapeDtypeStruct((BATCH, DIM), jnp.float32),
    mesh=plsc.VectorSubcoreMesh(core_axis_name="c", subcore_axis_name="s"),
    compiler_params=pltpu.CompilerParams(use_tc_tiling_on_sc=False),
    scratch_types=(
        [pltpu.VMEM((ROWS_PER_WORKER,), jnp.int32)]
        + [pltpu.VMEM((CHUNK, DIM), jnp.float32) for _ in range(NBUF)]
        + [pltpu.SemaphoreType.DMA for _ in range(2 * NBUF)]
    ),
)
def _gather_kernel(table_hbm, idx_hbm, out_hbm, idx_all, *bufs_and_sems):
    rows = bufs_and_sems[:NBUF]
    gsem = bufs_and_sems[NBUF:2 * NBUF]
    wsem = bufs_and_sems[2 * NBUF:]

    wid = lax.axis_index("s") * NUM_CORES + lax.axis_index("c")
    base = wid * ROWS_PER_WORKER
    pltpu.sync_copy(idx_hbm.at[pl.ds(base, ROWS_PER_WORKER)], idx_all)

    def start_gather(g, b):
        pltpu.async_copy(
            table_hbm.at[idx_all.at[pl.ds(g * CHUNK, CHUNK)]], rows[b], gsem[b])

    def wait_gather(g, b):
        pltpu.make_async_copy(
            table_hbm.at[idx_all.at[pl.ds(g * CHUNK, CHUNK)]], rows[b], gsem[b]).wait()

    def start_writeback(g, b):
        pltpu.async_copy(rows[b], out_hbm.at[pl.ds(base + g * CHUNK, CHUNK)], wsem[b])

    def wait_writeback(g, b):
        pltpu.make_async_copy(
            rows[b], out_hbm.at[pl.ds(base + g * CHUNK, CHUNK)], wsem[b]).wait()

    for b in range(NBUF):
        start_gather(b, b)
        if b >= SHIFT:
            wait_gather(b - SHIFT, b - SHIFT)
            start_writeback(b - SHIFT, b - SHIFT)

    def round_body(i, carry):
        g0 = i * NBUF
        for b in range(NBUF):
            g = g0 + b
            wait_writeback(g - NBUF, b)
            start_gather(g, b)
            bp = (b - SHIFT) % NBUF
            wait_gather(g - SHIFT, bp)
            start_writeback(g - SHIFT, bp)
        return carry

    lax.fori_loop(1, NUM_ROUNDS, round_body, 0)

    for j in range(SHIFT):
        g = NUM_CHUNKS - SHIFT + j
        b = g % NBUF
        wait_gather(g, b)
        start_writeback(g, b)
    for b in range(NBUF):
        g = NUM_CHUNKS - NBUF + b
        wait_writeback(g, b)


def kernel(token_ids, weights):
    r = token_ids.reshape(-1).astype(jnp.int32)
    idx = 2 * (r % HALF) + r // HALF
    packed = _repack(weights.T, weights.T)
    table_lin = packed.reshape(2 * HALF, DIM)
    out = _gather_kernel(table_lin, idx)
    return out.reshape(token_ids.shape + (DIM,))


# speedup vs baseline: 12.8068x; 1.2836x over previous
import functools

import jax
import jax.numpy as jnp
from jax import lax
from jax.experimental import pallas as pl
from jax.experimental.pallas import tpu as pltpu
from jax.experimental.pallas import tpu_sc as plsc

DIM = 64
BATCH = 16384 * 50
NROWS = 1000000

_info = plsc.get_sparse_core_info()
NUM_CORES = _info.num_cores
NUM_SUBCORES = _info.num_subcores
NUM_WORKERS = NUM_CORES * NUM_SUBCORES

ROWS_PER_WORKER = BATCH // NUM_WORKERS
CHUNK = 400
NUM_CHUNKS = ROWS_PER_WORKER // CHUNK
NBUF = 4
SHIFT = 2
NUM_ROUNDS = NUM_CHUNKS // NBUF

TBLK = 4096
TGRID = 123
HALF = TBLK * TGRID


def _repack_body(in1_ref, in2_ref, out_ref):
    out_ref[:, 0:DIM] = in1_ref[...].T
    out_ref[:, DIM:128] = in2_ref[...].T


_repack = pl.pallas_call(
    _repack_body,
    grid=(TGRID,),
    in_specs=[
        pl.BlockSpec((DIM, TBLK), lambda i: (0, i)),
        pl.BlockSpec(
            (DIM, TBLK),
            lambda i: (0, jnp.minimum(i + TGRID, (NROWS - 1) // TBLK))),
    ],
    out_specs=pl.BlockSpec((TBLK, 128), lambda i: (i, 0)),
    out_shape=jax.ShapeDtypeStruct((HALF, 128), jnp.float32),
)


BBLK = 128
OGRID = 16384 // BBLK


def _unpack_body(in_ref, out_ref):
    x3 = pltpu.einshape("(br)q->brq", in_ref[...], r=25)
    for t in range(50):
        c, p = t // 2, t % 2
        out_ref[t] = x3[:, c, p * DIM:(p + 1) * DIM].T


_unpack = pl.pallas_call(
    _unpack_body,
    grid=(OGRID,),
    in_specs=[pl.BlockSpec((BBLK * 25, 128), lambda i: (i, 0))],
    out_specs=pl.BlockSpec((50, DIM, BBLK), lambda i: (0, 0, i)),
    out_shape=jax.ShapeDtypeStruct((50, DIM, 16384), jnp.float32),
)


@functools.partial(
    pl.kernel,
    out_type=jax.ShapeDtypeStruct((BATCH, DIM), jnp.float32),
    mesh=plsc.VectorSubcoreMesh(core_axis_name="c", subcore_axis_name="s"),
    compiler_params=pltpu.CompilerParams(use_tc_tiling_on_sc=False),
    scratch_types=(
        [pltpu.VMEM((ROWS_PER_WORKER,), jnp.int32)]
        + [pltpu.VMEM((CHUNK, DIM), jnp.float32) for _ in range(NBUF)]
        + [pltpu.SemaphoreType.DMA for _ in range(2 * NBUF)]
    ),
)
def _gather_kernel(table_hbm, idx_hbm, out_hbm, idx_all, *bufs_and_sems):
    rows = bufs_and_sems[:NBUF]
    gsem = bufs_and_sems[NBUF:2 * NBUF]
    wsem = bufs_and_sems[2 * NBUF:]

    wid = lax.axis_index("s") * NUM_CORES + lax.axis_index("c")
    base = wid * ROWS_PER_WORKER
    pltpu.sync_copy(idx_hbm.at[pl.ds(base, ROWS_PER_WORKER)], idx_all)

    def start_gather(g, b):
        pltpu.async_copy(
            table_hbm.at[idx_all.at[pl.ds(g * CHUNK, CHUNK)]], rows[b], gsem[b])

    def wait_gather(g, b):
        pltpu.make_async_copy(
            table_hbm.at[idx_all.at[pl.ds(g * CHUNK, CHUNK)]], rows[b], gsem[b]).wait()

    def start_writeback(g, b):
        pltpu.async_copy(rows[b], out_hbm.at[pl.ds(base + g * CHUNK, CHUNK)], wsem[b])

    def wait_writeback(g, b):
        pltpu.make_async_copy(
            rows[b], out_hbm.at[pl.ds(base + g * CHUNK, CHUNK)], wsem[b]).wait()

    for b in range(NBUF):
        start_gather(b, b)
        if b >= SHIFT:
            wait_gather(b - SHIFT, b - SHIFT)
            start_writeback(b - SHIFT, b - SHIFT)

    def round_body(i, carry):
        g0 = i * NBUF
        for b in range(NBUF):
            g = g0 + b
            wait_writeback(g - NBUF, b)
            start_gather(g, b)
            bp = (b - SHIFT) % NBUF
            wait_gather(g - SHIFT, bp)
            start_writeback(g - SHIFT, bp)
        return carry

    lax.fori_loop(1, NUM_ROUNDS, round_body, 0)

    for j in range(SHIFT):
        g = NUM_CHUNKS - SHIFT + j
        b = g % NBUF
        wait_gather(g, b)
        start_writeback(g, b)
    for b in range(NBUF):
        g = NUM_CHUNKS - NBUF + b
        wait_writeback(g, b)


def kernel(token_ids, weights):
    r = token_ids.reshape(-1).astype(jnp.int32)
    idx = 2 * (r % HALF) + r // HALF
    packed = _repack(weights.T, weights.T)
    table_lin = packed.reshape(2 * HALF, DIM)
    out = _gather_kernel(table_lin, idx)
    out_phys = _unpack(out.reshape(BATCH * DIM // 128, 128))
    return jnp.transpose(out_phys, (2, 0, 1))


# speedup vs baseline: 13.3455x; 1.0421x over previous
import functools

import jax
import jax.numpy as jnp
from jax import lax
from jax.experimental import pallas as pl
from jax.experimental.pallas import tpu as pltpu
from jax.experimental.pallas import tpu_sc as plsc

DIM = 64
BATCH = 16384 * 50
NROWS = 1000000

_info = plsc.get_sparse_core_info()
NUM_CORES = _info.num_cores
NUM_SUBCORES = _info.num_subcores
NUM_WORKERS = NUM_CORES * NUM_SUBCORES

ROWS_PER_WORKER = BATCH // NUM_WORKERS
CHUNK = 400
NUM_CHUNKS = ROWS_PER_WORKER // CHUNK
NBUF = 4
SHIFT = 2
NUM_ROUNDS = NUM_CHUNKS // NBUF

TBLK = 8192
TGRID = 62
HALF = TBLK * TGRID


def _repack_body(in1_ref, in2_ref, out_ref):
    out_ref[:, 0:DIM] = in1_ref[...].T
    out_ref[:, DIM:128] = in2_ref[...].T


_repack = pl.pallas_call(
    _repack_body,
    grid=(TGRID,),
    in_specs=[
        pl.BlockSpec((DIM, TBLK), lambda i: (0, i)),
        pl.BlockSpec(
            (DIM, TBLK),
            lambda i: (0, jnp.minimum(i + TGRID, (NROWS - 1) // TBLK))),
    ],
    out_specs=pl.BlockSpec((TBLK, 128), lambda i: (i, 0)),
    out_shape=jax.ShapeDtypeStruct((HALF, 128), jnp.float32),
)


BBLK = 256
OGRID = 16384 // BBLK


def _unpack_body(in_ref, out_ref):
    x3 = pltpu.einshape("(br)q->brq", in_ref[...], r=25)
    for t in range(50):
        c, p = t // 2, t % 2
        out_ref[t] = x3[:, c, p * DIM:(p + 1) * DIM].T


_unpack = pl.pallas_call(
    _unpack_body,
    grid=(OGRID,),
    in_specs=[pl.BlockSpec((BBLK * 25, 128), lambda i: (i, 0))],
    out_specs=pl.BlockSpec((50, DIM, BBLK), lambda i: (0, 0, i)),
    out_shape=jax.ShapeDtypeStruct((50, DIM, 16384), jnp.float32),
)


@functools.partial(
    pl.kernel,
    out_type=jax.ShapeDtypeStruct((BATCH, DIM), jnp.float32),
    mesh=plsc.VectorSubcoreMesh(core_axis_name="c", subcore_axis_name="s"),
    compiler_params=pltpu.CompilerParams(use_tc_tiling_on_sc=False),
    scratch_types=(
        [pltpu.VMEM((ROWS_PER_WORKER,), jnp.int32)]
        + [pltpu.VMEM((CHUNK, DIM), jnp.float32) for _ in range(NBUF)]
        + [pltpu.SemaphoreType.DMA for _ in range(2 * NBUF)]
    ),
)
def _gather_kernel(table_hbm, idx_hbm, out_hbm, idx_all, *bufs_and_sems):
    rows = bufs_and_sems[:NBUF]
    gsem = bufs_and_sems[NBUF:2 * NBUF]
    wsem = bufs_and_sems[2 * NBUF:]

    wid = lax.axis_index("s") * NUM_CORES + lax.axis_index("c")
    base = wid * ROWS_PER_WORKER
    pltpu.sync_copy(idx_hbm.at[pl.ds(base, ROWS_PER_WORKER)], idx_all)

    def start_gather(g, b):
        pltpu.async_copy(
            table_hbm.at[idx_all.at[pl.ds(g * CHUNK, CHUNK)]], rows[b], gsem[b])

    def wait_gather(g, b):
        pltpu.make_async_copy(
            table_hbm.at[idx_all.at[pl.ds(g * CHUNK, CHUNK)]], rows[b], gsem[b]).wait()

    def start_writeback(g, b):
        pltpu.async_copy(rows[b], out_hbm.at[pl.ds(base + g * CHUNK, CHUNK)], wsem[b])

    def wait_writeback(g, b):
        pltpu.make_async_copy(
            rows[b], out_hbm.at[pl.ds(base + g * CHUNK, CHUNK)], wsem[b]).wait()

    for b in range(NBUF):
        start_gather(b, b)
        if b >= SHIFT:
            wait_gather(b - SHIFT, b - SHIFT)
            start_writeback(b - SHIFT, b - SHIFT)

    def round_body(i, carry):
        g0 = i * NBUF
        for b in range(NBUF):
            g = g0 + b
            wait_writeback(g - NBUF, b)
            start_gather(g, b)
            bp = (b - SHIFT) % NBUF
            wait_gather(g - SHIFT, bp)
            start_writeback(g - SHIFT, bp)
        return carry

    lax.fori_loop(1, NUM_ROUNDS, round_body, 0)

    for j in range(SHIFT):
        g = NUM_CHUNKS - SHIFT + j
        b = g % NBUF
        wait_gather(g, b)
        start_writeback(g, b)
    for b in range(NBUF):
        g = NUM_CHUNKS - NBUF + b
        wait_writeback(g, b)


def kernel(token_ids, weights):
    r = token_ids.reshape(-1).astype(jnp.int32)
    idx = 2 * (r % HALF) + r // HALF
    packed = _repack(weights.T, weights.T)
    table_lin = packed.reshape(2 * HALF, DIM)
    out = _gather_kernel(table_lin, idx)
    out_phys = _unpack(out.reshape(BATCH * DIM // 128, 128))
    return jnp.transpose(out_phys, (2, 0, 1))


# speedup vs baseline: 13.3468x; 1.0001x over previous
import functools

import jax
import jax.numpy as jnp
from jax import lax
from jax.experimental import pallas as pl
from jax.experimental.pallas import tpu as pltpu
from jax.experimental.pallas import tpu_sc as plsc

DIM = 64
BATCH = 16384 * 50
NROWS = 1000000

_info = plsc.get_sparse_core_info()
NUM_CORES = _info.num_cores
NUM_SUBCORES = _info.num_subcores
NUM_WORKERS = NUM_CORES * NUM_SUBCORES

ROWS_PER_WORKER = BATCH // NUM_WORKERS
CHUNK = 400
NUM_CHUNKS = ROWS_PER_WORKER // CHUNK
NBUF = 4
SHIFT = 2
NUM_ROUNDS = NUM_CHUNKS // NBUF

TBLK = 8192
TGRID = 62
HALF = TBLK * TGRID


def _repack_body(in1_ref, in2_ref, out_ref):
    out_ref[:, 0:DIM] = in1_ref[...].T
    out_ref[:, DIM:128] = in2_ref[...].T


_repack = pl.pallas_call(
    _repack_body,
    grid=(TGRID,),
    in_specs=[
        pl.BlockSpec((DIM, TBLK), lambda i: (0, i)),
        pl.BlockSpec(
            (DIM, TBLK),
            lambda i: (0, jnp.minimum(i + TGRID, (NROWS - 1) // TBLK))),
    ],
    out_specs=pl.BlockSpec((TBLK, 128), lambda i: (i, 0)),
    out_shape=jax.ShapeDtypeStruct((HALF, 128), jnp.float32),
)


BBLK = 512
OGRID = 16384 // BBLK


def _unpack_body(in_ref, out_ref):
    x3 = pltpu.einshape("(br)q->brq", in_ref[...], r=25)
    for t in range(50):
        c, p = t // 2, t % 2
        out_ref[t] = x3[:, c, p * DIM:(p + 1) * DIM].T


_unpack = pl.pallas_call(
    _unpack_body,
    grid=(OGRID,),
    in_specs=[pl.BlockSpec((BBLK * 25, 128), lambda i: (i, 0))],
    out_specs=pl.BlockSpec((50, DIM, BBLK), lambda i: (0, 0, i)),
    out_shape=jax.ShapeDtypeStruct((50, DIM, 16384), jnp.float32),
)


@functools.partial(
    pl.kernel,
    out_type=jax.ShapeDtypeStruct((BATCH, DIM), jnp.float32),
    mesh=plsc.VectorSubcoreMesh(core_axis_name="c", subcore_axis_name="s"),
    compiler_params=pltpu.CompilerParams(use_tc_tiling_on_sc=False),
    scratch_types=(
        [pltpu.VMEM((ROWS_PER_WORKER,), jnp.int32)]
        + [pltpu.VMEM((CHUNK, DIM), jnp.float32) for _ in range(NBUF)]
        + [pltpu.SemaphoreType.DMA for _ in range(2 * NBUF)]
    ),
)
def _gather_kernel(table_hbm, idx_hbm, out_hbm, idx_all, *bufs_and_sems):
    rows = bufs_and_sems[:NBUF]
    gsem = bufs_and_sems[NBUF:2 * NBUF]
    wsem = bufs_and_sems[2 * NBUF:]

    wid = lax.axis_index("s") * NUM_CORES + lax.axis_index("c")
    base = wid * ROWS_PER_WORKER
    pltpu.sync_copy(idx_hbm.at[pl.ds(base, ROWS_PER_WORKER)], idx_all)

    def start_gather(g, b):
        pltpu.async_copy(
            table_hbm.at[idx_all.at[pl.ds(g * CHUNK, CHUNK)]], rows[b], gsem[b])

    def wait_gather(g, b):
        pltpu.make_async_copy(
            table_hbm.at[idx_all.at[pl.ds(g * CHUNK, CHUNK)]], rows[b], gsem[b]).wait()

    def start_writeback(g, b):
        pltpu.async_copy(rows[b], out_hbm.at[pl.ds(base + g * CHUNK, CHUNK)], wsem[b])

    def wait_writeback(g, b):
        pltpu.make_async_copy(
            rows[b], out_hbm.at[pl.ds(base + g * CHUNK, CHUNK)], wsem[b]).wait()

    for b in range(NBUF):
        start_gather(b, b)
        if b >= SHIFT:
            wait_gather(b - SHIFT, b - SHIFT)
            start_writeback(b - SHIFT, b - SHIFT)

    def round_body(i, carry):
        g0 = i * NBUF
        for b in range(NBUF):
            g = g0 + b
            wait_writeback(g - NBUF, b)
            start_gather(g, b)
            bp = (b - SHIFT) % NBUF
            wait_gather(g - SHIFT, bp)
            start_writeback(g - SHIFT, bp)
        return carry

    lax.fori_loop(1, NUM_ROUNDS, round_body, 0)

    for j in range(SHIFT):
        g = NUM_CHUNKS - SHIFT + j
        b = g % NBUF
        wait_gather(g, b)
        start_writeback(g, b)
    for b in range(NBUF):
        g = NUM_CHUNKS - NBUF + b
        wait_writeback(g, b)


def kernel(token_ids, weights):
    r = token_ids.reshape(-1).astype(jnp.int32)
    idx = 2 * (r % HALF) + r // HALF
    packed = _repack(weights.T, weights.T)
    table_lin = packed.reshape(2 * HALF, DIM)
    out = _gather_kernel(table_lin, idx)
    out_phys = _unpack(out.reshape(BATCH * DIM // 128, 128))
    return jnp.transpose(out_phys, (2, 0, 1))
